# Initial kernel scaffold; baseline (speedup 1.0000x reference)
import jax
import jax.numpy as jnp
from jax.experimental import pallas as pl


def kernel(state, dropout, wr_w1, wr_b1, wr_w2, wr_b2, wh_w1, wh_b1, wh_w2, wh_b2, c1_rel_w, c1_rel_b, c1_root_w, c2_rel_w, c2_rel_b, c2_root_w, v_w1, v_b1, v_w2, v_b2, v_w3, v_b3):
    raise NotImplementedError("write your pallas kernel here")


# speedup vs baseline: 160.5898x; 160.5898x over previous
import jax
import jax.numpy as jnp
from jax.experimental import pallas as pl

_B = 64
_N = 128
_R = _B * _N


def _fwd(self_s, hum, wr1, wrb1, wr2, wrb2, wh1, whb1, wh2, whb2,
         comb1, rel1, c1b, comb2, rel2, c2b,
         vw1, vb1, vw2, vb2, vw3, vb3, out):
    f32 = jnp.float32
    dot = lambda a, b: jnp.dot(a, b, preferred_element_type=f32)
    relu = jax.nn.relu

    r = relu(dot(relu(dot(self_s[:], wr1[:]) + wrb1[:]), wr2[:]) + wrb2[:])

    hf = relu(dot(relu(dot(hum[:], wh1[:]) + whb1[:]), wh2[:]) + whb2[:])

    ridx = jax.lax.broadcasted_iota(jnp.int32, (_R, 1), 0)
    keep = (ridx & (_N - 1)) != (_N - 1)

    hm = jnp.where(keep, hf, 0.0)
    s1 = hm.reshape(_B, _N, hf.shape[-1]).sum(axis=1) + r

    t1 = dot(s1, rel1[:]) + c1b[:]
    x1r = relu(dot(r, comb1[:]) + t1)
    x1h = relu((dot(hf, comb1[:])).reshape(_B, _N, -1)
               + t1[:, None, :])
    x1m = jnp.where(keep.reshape(_B, _N, 1), x1h, 0.0)
    s2 = x1m.sum(axis=1) + x1r

    x2 = relu(dot(x1r, comb2[:]) + dot(s2, rel2[:]) + c2b[:])

    v = relu(dot(x2, vw1[:]) + vb1[:])
    v = relu(dot(v, vw2[:]) + vb2[:])
    out[:] = dot(v, vw3[:]) + vb3[:]


def kernel(state, dropout, wr_w1, wr_b1, wr_w2, wr_b2, wh_w1, wh_b1, wh_w2,
           wh_b2, c1_rel_w, c1_rel_b, c1_root_w, c2_rel_w, c2_rel_b,
           c2_root_w, v_w1, v_b1, v_w2, v_b2, v_w3, v_b3):
    f32 = jnp.float32
    B, A, _ = state.shape

    self_s = state[:, 0, :6]
    hum = state[:, :, 6:]
    hum = jnp.pad(hum, ((0, 0), (0, _N - A), (0, 0)))
    hum = hum.reshape(B * _N, 7)

    row = lambda b: b.reshape(1, -1)
    args = (
        self_s, hum,
        wr_w1.T, row(wr_b1), wr_w2.T, row(wr_b2),
        wh_w1.T, row(wh_b1), wh_w2.T, row(wh_b2),
        (c1_root_w - c1_rel_w).T, c1_rel_w.T, row(c1_rel_b),
        (c2_root_w - c2_rel_w).T, c2_rel_w.T, row(c2_rel_b),
        v_w1.T, row(v_b1), v_w2.T, row(v_b2), v_w3.T, row(v_b3),
    )
    return pl.pallas_call(
        _fwd,
        out_shape=jax.ShapeDtypeStruct((B, 1), f32),
    )(*args)
